# baseline (speedup 1.0000x reference)
import functools

import jax
import jax.numpy as jnp
from jax import lax
from jax.experimental import pallas as pl
from jax.experimental.pallas import tpu as pltpu
from jax.experimental.pallas import tpu_sc as plsc

BATCH = 64
SEQ = 512
EMBD = 512
NW = 32
PW = SEQ // NW
LANES = 16
VECS = EMBD // LANES
CR = 32
NCH = PW * BATCH // CR
NB = 6
GA = 3


def _emb_body(x_hbm, tok_hbm, pos_hbm, out_hbm,
              pos_v, tokid_v, oidx2_v, oidxf_v,
              rows0, rows1, rows2, rows3, rows4, rows5, psem,
              gsem0, gsem1, gsem2, gsem3, gsem4, gsem5,
              ssem0, ssem1, ssem2, ssem3, ssem4, ssem5):
    wid = lax.axis_index("s") * 2 + lax.axis_index("c")
    p0 = wid * PW
    ph = pltpu.async_copy(pos_hbm.at[pl.ds(p0, PW)], pos_v, psem)

    bvec = lax.iota(jnp.int32, LANES) * SEQ
    for j in range(PW):
        for kk in range(BATCH // LANES):
            val = bvec + (kk * LANES * SEQ + p0 + j)
            t = j * 2 + kk // 2
            oidx2_v[t, pl.ds((kk % 2) * LANES, LANES)] = val
            oidxf_v[pl.ds(j * BATCH + kk * LANES, LANES)] = val
    pltpu.sync_copy(x_hbm.at[oidxf_v], tokid_v)

    rows = (rows0, rows1, rows2, rows3, rows4, rows5)
    gsem = (gsem0, gsem1, gsem2, gsem3, gsem4, gsem5)
    ssem = (ssem0, ssem1, ssem2, ssem3, ssem4, ssem5)

    def start_gather(t, buf):
        return pltpu.async_copy(
            tok_hbm.at[tokid_v.at[pl.ds(t * CR, CR)]], rows[buf],
            gsem[buf])

    def add_pos(t, buf):
        r = rows[buf]
        j = t // 2
        pv = [pos_v[j, pl.ds(k * LANES, LANES)] for k in range(VECS)]

        def body(row, carry):
            for k in range(VECS):
                sl = pl.ds(k * LANES, LANES)
                r[row, sl] = r[row, sl] + pv[k]
            return carry

        lax.fori_loop(0, CR, body, 0)

    def start_scatter(t, buf):
        return pltpu.async_copy(rows[buf], out_hbm.at[oidx2_v.at[t]],
                                ssem[buf])

    g = [None] * NCH
    s = [None] * NCH
    for t in range(GA):
        g[t] = start_gather(t, t % NB)
    ph.wait()
    for t in range(NCH):
        buf = t % NB
        if t + GA < NCH:
            if t + GA - NB >= 0:
                s[t + GA - NB].wait()
            g[t + GA] = start_gather(t + GA, (t + GA) % NB)
        g[t].wait()
        add_pos(t, buf)
        s[t] = start_scatter(t, buf)
    for t in range(NCH - NB + GA, NCH):
        s[t].wait()


def kernel(x, token_table, position_table):
    xf = x.reshape(-1).astype(jnp.int32)
    mesh = plsc.VectorSubcoreMesh(core_axis_name="c", subcore_axis_name="s")
    f = functools.partial(
        pl.kernel,
        mesh=mesh,
        out_type=jax.ShapeDtypeStruct((BATCH * SEQ, EMBD), jnp.float32),
        scratch_types=[
            pltpu.VMEM((PW, EMBD), jnp.float32),
            pltpu.VMEM((PW * BATCH,), jnp.int32),
            pltpu.VMEM((NCH, CR), jnp.int32),
            pltpu.VMEM((PW * BATCH,), jnp.int32),
            pltpu.VMEM((CR, EMBD), jnp.float32),
            pltpu.VMEM((CR, EMBD), jnp.float32),
            pltpu.VMEM((CR, EMBD), jnp.float32),
            pltpu.VMEM((CR, EMBD), jnp.float32),
            pltpu.VMEM((CR, EMBD), jnp.float32),
            pltpu.VMEM((CR, EMBD), jnp.float32),
            pltpu.SemaphoreType.DMA,
            pltpu.SemaphoreType.DMA,
            pltpu.SemaphoreType.DMA,
            pltpu.SemaphoreType.DMA,
            pltpu.SemaphoreType.DMA,
            pltpu.SemaphoreType.DMA,
            pltpu.SemaphoreType.DMA,
            pltpu.SemaphoreType.DMA,
            pltpu.SemaphoreType.DMA,
            pltpu.SemaphoreType.DMA,
            pltpu.SemaphoreType.DMA,
            pltpu.SemaphoreType.DMA,
            pltpu.SemaphoreType.DMA,
        ],
    )(_emb_body)
    out = f(xf, token_table, position_table)
    return out.reshape(BATCH, SEQ, EMBD)
